# baseline (speedup 1.0000x reference)
import functools

import jax
import jax.numpy as jnp
from jax import lax
from jax.experimental import pallas as pl
from jax.experimental.pallas import tpu as pltpu
from jax.experimental.pallas import tpu_sc as plsc

B = 16384
D = 64
NC = 2
NS = 16
NW = NC * NS
BPW = B // NW
CH = 128
NCHUNK = BPW // CH
LANES = 16
VPR = D // LANES

_SIN_COEFFS = (
    -1.0 / 1307674368000.0,
    1.0 / 6227020800.0,
    -1.0 / 39916800.0,
    1.0 / 362880.0,
    -1.0 / 5040.0,
    1.0 / 120.0,
    -1.0 / 6.0,
    1.0,
)
_COS_COEFFS = (
    -1.0 / 87178291200.0,
    1.0 / 479001600.0,
    -1.0 / 3628800.0,
    1.0 / 40320.0,
    -1.0 / 720.0,
    1.0 / 24.0,
    -0.5,
    1.0,
)


def _sincos(x):
    z = x * x
    s = jnp.float32(_SIN_COEFFS[0])
    for c in _SIN_COEFFS[1:]:
        s = s * z + jnp.float32(c)
    s = s * x
    c_acc = jnp.float32(_COS_COEFFS[0])
    for c in _COS_COEFFS[1:]:
        c_acc = c_acc * z + jnp.float32(c)
    return s, c_acc


_mesh = plsc.VectorSubcoreMesh(core_axis_name="c", subcore_axis_name="s")


@functools.partial(
    pl.kernel,
    mesh=_mesh,
    compiler_params=pltpu.CompilerParams(use_tc_tiling_on_sc=False),
    out_type=(
        jax.ShapeDtypeStruct((B, D), jnp.float32),
        jax.ShapeDtypeStruct((B, D), jnp.float32),
    ),
    scratch_types=[
        pltpu.VMEM((BPW,), jnp.int32),
        pltpu.VMEM((BPW,), jnp.int32),
        pltpu.VMEM((BPW, D), jnp.float32),
        pltpu.VMEM((BPW, D), jnp.float32),
        pltpu.VMEM((BPW, D), jnp.float32),
        pltpu.SemaphoreType.DMA,
    ],
)
def _rotate_sc(e1_hbm, r_hbm, er_hbm, ei_hbm, rel_hbm, outr_hbm, outi_hbm,
               idx1_v, idx2_v, er_v, ei_v, th_v, sem):
    wid = lax.axis_index("s") * NC + lax.axis_index("c")
    base = wid * BPW

    pltpu.sync_copy(e1_hbm.at[pl.ds(base, BPW)], idx1_v)
    pltpu.sync_copy(r_hbm.at[pl.ds(base, BPW)], idx2_v)

    copies = []
    for c in range(NCHUNK):
        sl = pl.ds(c * CH, CH)
        copies.append(pltpu.async_copy(er_hbm.at[idx1_v.at[sl]], er_v.at[sl], sem))
        copies.append(pltpu.async_copy(ei_hbm.at[idx1_v.at[sl]], ei_v.at[sl], sem))
        copies.append(pltpu.async_copy(rel_hbm.at[idx2_v.at[sl]], th_v.at[sl], sem))
    for cp in copies:
        cp.wait()

    def row_body(i, _):
        for j in range(VPR):
            dsl = pl.ds(j * LANES, LANES)
            theta = th_v[i, dsl]
            s, c = _sincos(theta)
            a = er_v[i, dsl]
            b = ei_v[i, dsl]
            er_v[i, dsl] = a * c - b * s
            ei_v[i, dsl] = a * s + b * c
        return _

    lax.fori_loop(0, BPW, row_body, None)

    pltpu.sync_copy(er_v, outr_hbm.at[pl.ds(base, BPW)])
    pltpu.sync_copy(ei_v, outi_hbm.at[pl.ds(base, BPW)])


def kernel(e1, r, entity_real, entity_img, relation):
    e1 = e1.astype(jnp.int32)
    r = r.astype(jnp.int32)
    return _rotate_sc(e1, r, entity_real, entity_img, relation)
